# baseline (speedup 1.0000x reference)
import functools

import jax
import jax.numpy as jnp
from jax import lax
from jax.experimental import pallas as pl
from jax.experimental.pallas import tpu as pltpu
from jax.experimental.pallas import tpu_sc as plsc

_NC = 2
_NS = 16
_L = 16
_FW = 8
_BISECT = {"syncpipe", "zero", "compute"}
_SC = 80
_PIECES = [(o, min(128, _SC - o)) for o in range(0, _SC, 128)]


def _gath(v16, pat):
    return v16.at[pat].get(mode="promise_in_bounds")


def _sc_mesh():
    return plsc.VectorSubcoreMesh(core_axis_name="c", subcore_axis_name="s")


def _edge_sc_build(N, KK, two, E):
    EHS = _NS // KK
    EC = E // _NC
    ET = EC // EHS
    NSC = ET // _SC
    assert ET % _SC == 0 and NSC % 2 == 0

    def slot_types():
        t = [
            pltpu.VMEM((_SC,), jnp.int32),
            pltpu.VMEM((_SC,), jnp.int32),
            pltpu.VMEM((_SC,), jnp.float32),
            pltpu.VMEM((_SC,), jnp.int32),
            pltpu.VMEM((_SC, _FW), jnp.float32),
            pltpu.SemaphoreType.DMA,
            pltpu.SemaphoreType.DMA,
        ]
        if two:
            t += [
                pltpu.VMEM((_SC,), jnp.int32),
                pltpu.VMEM((_SC, _FW), jnp.float32),
            ]
        return t

    nslot = len(slot_types())
    scratch = [pltpu.VMEM((N, _FW), jnp.float32)] + slot_types() + slot_types()

    @functools.partial(
        pl.kernel,
        out_type=jax.ShapeDtypeStruct((_NC * _NS * N, _FW), jnp.float32),
        mesh=_sc_mesh(),
        scratch_types=scratch,
        compiler_params=pltpu.CompilerParams(
            use_tc_tiling_on_sc=False, needs_layout_passes=False),
    )
    def body(table_hbm, src_hbm, dst_hbm, adj_hbm, out_hbm, acc, *slots):
        cid = lax.axis_index("c")
        sid = lax.axis_index("s")
        fg = sid % KK
        eh = sid // KK
        ebase = cid * EC + eh * ET
        fgv = jnp.full((_L,), fg, jnp.int32)
        zero16 = jnp.zeros((_L,), jnp.float32)
        lane = lax.broadcasted_iota(jnp.int32, (_L,), 0)
        COL8 = lane % _FW
        PAT0 = lane // _FW
        PATS = [PAT0 + 2 * p for p in range(_L // 2)]

        def S(slot):
            return slots[slot * nslot:(slot + 1) * nslot]

        if "zero" in _BISECT:
            def zbody(r, carry):
                rvec = jnp.full((_L,), 2 * r, jnp.int32) + PAT0
                plsc.store_scatter(acc, [rvec, COL8], zero16)
                return carry
            lax.fori_loop(0, N // 2, zbody, 0)

        def load_idx(slot, ci):
            sv, dv, av = S(slot)[0], S(slot)[1], S(slot)[2]
            isem = S(slot)[5]
            off = ebase + ci * _SC
            pltpu.async_copy(src_hbm.at[pl.ds(off, _SC)], sv, isem)
            pltpu.async_copy(dst_hbm.at[pl.ds(off, _SC)], dv, isem)
            pltpu.async_copy(adj_hbm.at[pl.ds(off, _SC)], av, isem)

        def wait_idx(slot):
            sv, dv, av = S(slot)[0], S(slot)[1], S(slot)[2]
            isem = S(slot)[5]
            pltpu.make_async_copy(src_hbm.at[pl.ds(0, _SC)], sv, isem).wait()
            pltpu.make_async_copy(dst_hbm.at[pl.ds(0, _SC)], dv, isem).wait()
            pltpu.make_async_copy(adj_hbm.at[pl.ds(0, _SC)], av, isem).wait()

        def prep_fire(slot):
            s = S(slot)
            sv, dv, dvt, rows_d, gsem = s[0], s[1], s[3], s[4], s[6]

            def pbody(g, carry):
                b = g * _L
                dvt[pl.ds(b, _L)] = dv[pl.ds(b, _L)] * KK + fgv
                if two:
                    svt = s[7]
                    svt[pl.ds(b, _L)] = sv[pl.ds(b, _L)] * KK + fgv
                return carry
            lax.fori_loop(0, _SC // _L, pbody, 0)
            for (o, ln) in _PIECES:
                pltpu.async_copy(
                    table_hbm.at[dvt.at[pl.ds(o, ln)]],
                    rows_d.at[pl.ds(o, ln)], gsem)
            if two:
                svt, rows_s = s[7], s[8]
                for (o, ln) in _PIECES:
                    pltpu.async_copy(
                        table_hbm.at[svt.at[pl.ds(o, ln)]],
                        rows_s.at[pl.ds(o, ln)], gsem)

        def wait_gathers(slot):
            s = S(slot)
            dvt, rows_d, gsem = s[3], s[4], s[6]
            for (o, ln) in _PIECES:
                pltpu.make_async_copy(
                    table_hbm.at[dvt.at[pl.ds(o, ln)]],
                    rows_d.at[pl.ds(o, ln)], gsem).wait()
            if two:
                svt, rows_s = s[7], s[8]
                for (o, ln) in _PIECES:
                    pltpu.make_async_copy(
                        table_hbm.at[svt.at[pl.ds(o, ln)]],
                        rows_s.at[pl.ds(o, ln)], gsem).wait()

        def compute(slot):
            if "compute" not in _BISECT:
                return
            s = S(slot)
            sv, av, rows_d = s[0], s[2], s[4]
            rows_s = s[8] if two else None

            def gbody(g, carry):
                b = g * _L
                s16 = sv[pl.ds(b, _L)]
                a16 = av[pl.ds(b, _L)]
                bv = jnp.full((_L,), b, jnp.int32)
                for p in range(_L // 2):
                    pat = PATS[p]
                    ridx = bv + pat
                    srcp = _gath(s16, pat)
                    ap = _gath(a16, pat)
                    vd = plsc.load_gather(rows_d, [ridx, COL8])
                    if two:
                        vs = plsc.load_gather(rows_s, [ridx, COL8])
                        dd = vs - vd
                        val = dd * dd * ap
                    else:
                        val = vd * ap
                    plsc.addupdate_scatter(acc, [srcp, COL8], val)
                return carry
            lax.fori_loop(0, _SC // _L, gbody, 0)

        if "syncpipe" in _BISECT:
            def sloop(ci, carry):
                s = S(0)
                sv, dv, av, dvt, rows_d = s[0], s[1], s[2], s[3], s[4]
                off = ebase + ci * _SC
                pltpu.sync_copy(src_hbm.at[pl.ds(off, _SC)], sv)
                pltpu.sync_copy(dst_hbm.at[pl.ds(off, _SC)], dv)
                pltpu.sync_copy(adj_hbm.at[pl.ds(off, _SC)], av)

                def pbody(g, c2):
                    b = g * _L
                    dvt[pl.ds(b, _L)] = dv[pl.ds(b, _L)] * KK + fgv
                    if two:
                        svt = s[7]
                        svt[pl.ds(b, _L)] = sv[pl.ds(b, _L)] * KK + fgv
                    return c2
                lax.fori_loop(0, _SC // _L, pbody, 0)
                pltpu.sync_copy(table_hbm.at[dvt], rows_d)
                if two:
                    pltpu.sync_copy(table_hbm.at[s[7]], s[8])
                compute(0)
                return carry
            lax.fori_loop(0, NSC, sloop, 0)

        if "pipe" in _BISECT:
            load_idx(0, 0)
            wait_idx(0)
            prep_fire(0)

            def main(i2, carry):
                load_idx(1, 2 * i2 + 1)
                wait_gathers(0)
                wait_idx(1)
                prep_fire(1)
                compute(0)
                load_idx(0, 2 * i2 + 2)
                wait_gathers(1)
                wait_idx(0)
                prep_fire(0)
                compute(1)
                return carry
            lax.fori_loop(0, NSC // 2 - 1, main, 0)

            load_idx(1, NSC - 1)
            wait_gathers(0)
            wait_idx(1)
            prep_fire(1)
            compute(0)
            wait_gathers(1)
            compute(1)

        pltpu.sync_copy(acc, out_hbm.at[pl.ds((cid * _NS + sid) * N, N)])

    return body


def _to_partials(out, N, KK):
    EHS = _NS // KK
    p = out.reshape(_NC, EHS, KK, N, _FW)
    p = jnp.transpose(p, (0, 1, 3, 2, 4))
    return p.reshape(_NC * EHS, N, KK * _FW)


def _edge_mask_sc(x, src, dst, adj):
    N, F = x.shape
    KK = F // _FW
    xr = x.reshape(N * KK, _FW)
    out = _edge_sc_build(N, KK, True, src.shape[0])(xr, src, dst, adj)
    return _to_partials(out, N, KK)


def _propagate_sc(y, src, dst, adj):
    N, G = y.shape
    KK = G // _FW
    yr = y.reshape(N * KK, _FW)
    out = _edge_sc_build(N, KK, False, src.shape[0])(yr, src, dst, adj)
    return _to_partials(out, N, KK)


def _assemble(p_ref):
    t = p_ref[0]
    for p in range(1, p_ref.shape[0]):
        t = t + p_ref[p]
    return t


def _fc_tc(x, P, degcol, sigma, W, b):
    N, F = x.shape
    G = W.shape[1]

    def body(x_ref, p_ref, deg_ref, sig_ref, w_ref, b_ref, o_ref):
        msum = _assemble(p_ref)
        sig = sig_ref[...]
        inv = 1.0 / (sig * sig)
        t = msum * inv / deg_ref[...]
        xm = jnp.exp(-t) * x_ref[...]
        o_ref[...] = (
            jnp.dot(xm, w_ref[...], preferred_element_type=jnp.float32)
            + b_ref[...]
        )

    return pl.pallas_call(
        body, out_shape=jax.ShapeDtypeStruct((N, G), jnp.float32),
    )(x, P, degcol, sigma.reshape(1, F), W, b.reshape(1, G))


def _relu_combine_tc(P, N, G):
    def body(p_ref, o_ref):
        o_ref[...] = jnp.maximum(_assemble(p_ref), 0.0)

    return pl.pallas_call(
        body, out_shape=jax.ShapeDtypeStruct((N, G), jnp.float32),
    )(P)


def _log_softmax_tc(P, N, GP, nclass):
    def body(p_ref, o_ref):
        s = _assemble(p_ref)
        col = lax.broadcasted_iota(jnp.int32, (N, GP), 1)
        valid = col < nclass
        s = jnp.where(valid, s, -1e30)
        m = jnp.max(s, axis=1, keepdims=True)
        e = jnp.where(valid, jnp.exp(s - m), 0.0)
        lse = jnp.log(jnp.sum(e, axis=1, keepdims=True))
        r = s - m - lse
        o_ref[...] = r[:, :nclass]

    return pl.pallas_call(
        body, out_shape=jax.ShapeDtypeStruct((N, nclass), jnp.float32),
    )(P)


def kernel(x, edge_index, adj_vals, deg, sigma1, W1, b1, sigma2, W2, b2):
    N, F = x.shape
    src = edge_index[0]
    dst = edge_index[1]
    degcol = deg.reshape(N, 1)
    nclass = W2.shape[1]
    nhid = W1.shape[1]
    gp = ((nclass + _FW - 1) // _FW) * _FW
    while _NS % (gp // _FW) != 0:
        gp += _FW
    W2p = jnp.pad(W2, ((0, 0), (0, gp - nclass)))
    b2p = jnp.pad(b2, (0, gp - nclass))

    m1p = _edge_mask_sc(x, src, dst, adj_vals)
    y1 = _fc_tc(x, m1p, degcol, sigma1, W1, b1)
    o1p = _propagate_sc(y1, src, dst, adj_vals)
    h = _relu_combine_tc(o1p, N, nhid)
    m2p = _edge_mask_sc(h, src, dst, adj_vals)
    y2 = _fc_tc(h, m2p, degcol, sigma2, W2p, b2p)
    o2p = _propagate_sc(y2, src, dst, adj_vals)
    return _log_softmax_tc(o2p, N, gp, nclass)


# speedup vs baseline: 3.2623x; 3.2623x over previous
import functools

import jax
import jax.numpy as jnp
from jax import lax
from jax.experimental import pallas as pl
from jax.experimental.pallas import tpu as pltpu
from jax.experimental.pallas import tpu_sc as plsc

_NC = 2
_NS = 16
_L = 16
_SC = 640
_NP = 5
_PL = 128


def _gath(v16, pat):
    return v16.at[pat].get(mode="promise_in_bounds")


def _sc_mesh():
    return plsc.VectorSubcoreMesh(core_axis_name="c", subcore_axis_name="s")


def _edge_sc_build(N, KK, FW, two, E):
    assert KK == _NS
    EV = _L // FW
    EC = E // _NC
    ET = EC
    NSC = ET // _SC
    assert ET % _SC == 0

    def slot_types():
        t = [
            pltpu.VMEM((_SC,), jnp.int32),
            pltpu.VMEM((_SC,), jnp.int32),
            pltpu.VMEM((_SC,), jnp.float32),
            pltpu.VMEM((_NP, _PL), jnp.int32),
            pltpu.VMEM((_NP, _PL, FW), jnp.float32),
            pltpu.SemaphoreType.DMA,
            pltpu.SemaphoreType.DMA,
        ]
        if two:
            t += [
                pltpu.VMEM((_NP, _PL), jnp.int32),
                pltpu.VMEM((_NP, _PL, FW), jnp.float32),
            ]
        return t

    nslot = len(slot_types())
    scratch = [pltpu.VMEM((N, FW), jnp.float32)] + slot_types() + slot_types()

    @functools.partial(
        pl.kernel,
        out_type=jax.ShapeDtypeStruct((_NC * _NS * N, FW), jnp.float32),
        mesh=_sc_mesh(),
        scratch_types=scratch,
        compiler_params=pltpu.CompilerParams(
            use_tc_tiling_on_sc=False, needs_layout_passes=False),
    )
    def body(table_hbm, src_hbm, dst_hbm, adj_hbm, out_hbm, acc, *slots):
        cid = lax.axis_index("c")
        sid = lax.axis_index("s")
        fg = sid
        ebase = cid * EC
        fgv = jnp.full((_L,), fg, jnp.int32)
        zero16 = jnp.zeros((_L,), jnp.float32)
        lane = lax.broadcasted_iota(jnp.int32, (_L,), 0)
        COL = lane % FW
        PATB = lane // FW

        def S(slot):
            return slots[slot * nslot:(slot + 1) * nslot]

        def zbody(r, carry):
            rvec = jnp.full((_L,), EV * r, jnp.int32) + PATB
            plsc.store_scatter(acc, [rvec, COL], zero16)
            return carry
        lax.fori_loop(0, N // EV, zbody, 0)

        def load_idx(slot, ci):
            sv, dv, av = S(slot)[0], S(slot)[1], S(slot)[2]
            isem = S(slot)[5]
            off = ebase + ci * _SC
            pltpu.async_copy(src_hbm.at[pl.ds(off, _SC)], sv, isem)
            pltpu.async_copy(dst_hbm.at[pl.ds(off, _SC)], dv, isem)
            pltpu.async_copy(adj_hbm.at[pl.ds(off, _SC)], av, isem)

        def wait_idx(slot):
            sv, dv, av = S(slot)[0], S(slot)[1], S(slot)[2]
            isem = S(slot)[5]
            pltpu.make_async_copy(src_hbm.at[pl.ds(0, _SC)], sv, isem).wait()
            pltpu.make_async_copy(dst_hbm.at[pl.ds(0, _SC)], dv, isem).wait()
            pltpu.make_async_copy(adj_hbm.at[pl.ds(0, _SC)], av, isem).wait()

        def prep_fire(slot):
            s = S(slot)
            sv, dv, dvt, rows_d, gsem = s[0], s[1], s[3], s[4], s[6]
            for k in range(_NP):
                def pbody(g8, carry):
                    lb = g8 * _L
                    gb = k * _PL + lb
                    dvt[k, pl.ds(lb, _L)] = dv[pl.ds(gb, _L)] * KK + fgv
                    if two:
                        svt = s[7]
                        svt[k, pl.ds(lb, _L)] = sv[pl.ds(gb, _L)] * KK + fgv
                    return carry
                lax.fori_loop(0, _PL // _L, pbody, 0)
            for k in range(_NP):
                pltpu.async_copy(
                    table_hbm.at[dvt.at[k]], rows_d.at[k], gsem)
            if two:
                svt, rows_s = s[7], s[8]
                for k in range(_NP):
                    pltpu.async_copy(
                        table_hbm.at[svt.at[k]], rows_s.at[k], gsem)

        def wait_gathers(slot):
            s = S(slot)
            dvt, rows_d, gsem = s[3], s[4], s[6]
            for k in range(_NP):
                pltpu.make_async_copy(
                    table_hbm.at[dvt.at[k]], rows_d.at[k], gsem).wait()
            if two:
                svt, rows_s = s[7], s[8]
                for k in range(_NP):
                    pltpu.make_async_copy(
                        table_hbm.at[svt.at[k]], rows_s.at[k], gsem).wait()

        def compute(slot):
            s = S(slot)
            sv, av = s[0], s[2]
            rows_d = s[4]
            rows_s = s[8] if two else None
            for k in range(_NP):
                rd_k = rows_d.at[k]
                rs_k = rows_s.at[k] if two else None

                def gbody(g8, carry):
                    lb = g8 * _L
                    gb = k * _PL + lb
                    s16 = sv[pl.ds(gb, _L)]
                    a16 = av[pl.ds(gb, _L)]
                    lbv = jnp.full((_L,), lb, jnp.int32)
                    for p in range(_L // EV):
                        pat = PATB + EV * p
                        ridx = lbv + pat
                        srcp = _gath(s16, pat)
                        ap = _gath(a16, pat)
                        vd = plsc.load_gather(rd_k, [ridx, COL])
                        if two:
                            vs = plsc.load_gather(rs_k, [ridx, COL])
                            dd = vs - vd
                            val = dd * dd * ap
                        else:
                            val = vd * ap
                        plsc.addupdate_scatter(acc, [srcp, COL], val)
                    return carry
                lax.fori_loop(0, _PL // _L, gbody, 0)

        load_idx(0, 0)
        wait_idx(0)
        prep_fire(0)

        def main(i2, carry):
            load_idx(1, 2 * i2 + 1)
            wait_gathers(0)
            wait_idx(1)
            prep_fire(1)
            compute(0)
            load_idx(0, 2 * i2 + 2)
            wait_gathers(1)
            wait_idx(0)
            prep_fire(0)
            compute(1)
            return carry
        lax.fori_loop(0, NSC // 2, main, 0)

        wait_gathers(0)
        compute(0)

        pltpu.sync_copy(acc, out_hbm.at[pl.ds((cid * _NS + sid) * N, N)])

    return body


def _to_partials(out, N, FW):
    p = out.reshape(_NC, _NS, N, FW)
    p = jnp.transpose(p, (0, 2, 1, 3))
    return p.reshape(_NC, N, _NS * FW)


def _edge_mask_sc(x, src, dst, adj):
    N, F = x.shape
    if F < _NS * 8:
        x = jnp.pad(x, ((0, 0), (0, _NS * 8 - F)))
    FW = x.shape[1] // _NS
    xr = x.reshape(N * _NS, FW)
    out = _edge_sc_build(N, _NS, FW, True, src.shape[0])(xr, src, dst, adj)
    return _to_partials(out, N, FW)[:, :, :F]


def _propagate_sc(y, src, dst, adj):
    N, G = y.shape
    if G < _NS * 8:
        y = jnp.pad(y, ((0, 0), (0, _NS * 8 - G)))
    FW = y.shape[1] // _NS
    yr = y.reshape(N * _NS, FW)
    out = _edge_sc_build(N, _NS, FW, False, src.shape[0])(yr, src, dst, adj)
    return _to_partials(out, N, FW)[:, :, :G]


def _assemble(p_ref):
    t = p_ref[0]
    for p in range(1, p_ref.shape[0]):
        t = t + p_ref[p]
    return t


def _fc_tc(x, P, degcol, sigma, W, b):
    N, F = x.shape
    G = W.shape[1]

    def body(x_ref, p_ref, deg_ref, sig_ref, w_ref, b_ref, o_ref):
        msum = _assemble(p_ref)
        sig = sig_ref[...]
        inv = 1.0 / (sig * sig)
        t = msum * inv / deg_ref[...]
        xm = jnp.exp(-t) * x_ref[...]
        o_ref[...] = (
            jnp.dot(xm, w_ref[...], preferred_element_type=jnp.float32)
            + b_ref[...]
        )

    return pl.pallas_call(
        body, out_shape=jax.ShapeDtypeStruct((N, G), jnp.float32),
    )(x, P, degcol, sigma.reshape(1, F), W, b.reshape(1, G))


def _relu_combine_tc(P, N, G):
    def body(p_ref, o_ref):
        o_ref[...] = jnp.maximum(_assemble(p_ref), 0.0)

    return pl.pallas_call(
        body, out_shape=jax.ShapeDtypeStruct((N, G), jnp.float32),
    )(P)


def _log_softmax_tc(P, N, GP, nclass):
    def body(p_ref, o_ref):
        s = _assemble(p_ref)
        col = lax.broadcasted_iota(jnp.int32, (N, GP), 1)
        valid = col < nclass
        s = jnp.where(valid, s, -1e30)
        m = jnp.max(s, axis=1, keepdims=True)
        e = jnp.where(valid, jnp.exp(s - m), 0.0)
        lse = jnp.log(jnp.sum(e, axis=1, keepdims=True))
        r = s - m - lse
        o_ref[...] = r[:, :nclass]

    return pl.pallas_call(
        body, out_shape=jax.ShapeDtypeStruct((N, nclass), jnp.float32),
    )(P)


def kernel(x, edge_index, adj_vals, deg, sigma1, W1, b1, sigma2, W2, b2):
    N, F = x.shape
    src = edge_index[0]
    dst = edge_index[1]
    degcol = deg.reshape(N, 1)
    nclass = W2.shape[1]
    nhid = W1.shape[1]
    gp = _NS
    while gp < nclass or (_L % (gp // _NS)) != 0:
        gp *= 2
    W2p = jnp.pad(W2, ((0, 0), (0, gp - nclass)))
    b2p = jnp.pad(b2, (0, gp - nclass))

    m1p = _edge_mask_sc(x, src, dst, adj_vals)
    y1 = _fc_tc(x, m1p, degcol, sigma1, W1, b1)
    o1p = _propagate_sc(y1, src, dst, adj_vals)
    h = _relu_combine_tc(o1p, N, nhid)
    m2p = _edge_mask_sc(h, src, dst, adj_vals)
    y2 = _fc_tc(h, m2p, degcol, sigma2, W2p, b2p)
    o2p = _propagate_sc(y2, src, dst, adj_vals)
    return _log_softmax_tc(o2p, N, gp, nclass)


# speedup vs baseline: 7.9204x; 2.4278x over previous
import functools

import jax
import jax.numpy as jnp
from jax import lax
from jax.experimental import pallas as pl
from jax.experimental.pallas import tpu as pltpu
from jax.experimental.pallas import tpu_sc as plsc

_NC = 2
_NS = 16
_L = 16
_CH = 128


def _zero_chunk_rows(rpt):
    for z in range(min(rpt, 64), 0, -1):
        if rpt % z == 0:
            return z
    return 1


def _lane_bcast(v16, lane):
    sel = jnp.full((_L,), lane, jnp.int32)
    return v16.at[sel].get(mode="promise_in_bounds")


def _scale_groups(av_ref, n, blockfn):
    gfull, rem = n // _L, n % _L

    def group(g, en):
        av16 = av_ref[pl.ds(g * _L, _L)]
        for e16 in range(en):
            blockfn(g * _L + e16, _lane_bcast(av16, e16))

    if gfull:
        def gbody(g, carry):
            group(g, _L)
            return carry
        lax.fori_loop(0, gfull, gbody, 0)
    if rem:
        group(gfull, rem)


def _sc_mesh():
    return plsc.VectorSubcoreMesh(core_axis_name="c", subcore_axis_name="s")


def _edge_mask_sc(x, src, dst, adj):
    N, F = x.shape
    E = src.shape[0]
    EC = E // _NC
    ET = EC // _NS
    nfull = ET // _CH
    tail = ET % _CH
    NP = -(-N // (_NS * 128)) * (_NS * 128)
    RPT = NP // _NS
    ZB = _zero_chunk_rows(RPT)
    nf = F // _L

    scratch = [
        pltpu.VMEM_SHARED((NP, F), jnp.float32),
        pltpu.VMEM((_CH,), jnp.int32),
        pltpu.VMEM((_CH,), jnp.int32),
        pltpu.VMEM((_CH,), jnp.float32),
        pltpu.VMEM((_CH, F), jnp.float32),
        pltpu.VMEM((_CH, F), jnp.float32),
        pltpu.VMEM((ZB, F), jnp.float32),
    ]
    if tail:
        tpad = ((tail + _L - 1) // _L) * _L
        scratch += [
            pltpu.VMEM((tail,), jnp.int32),
            pltpu.VMEM((tail,), jnp.int32),
            pltpu.VMEM((tpad,), jnp.float32),
            pltpu.VMEM((tail, F), jnp.float32),
            pltpu.VMEM((tail, F), jnp.float32),
        ]

    @functools.partial(
        pl.kernel,
        out_type=jax.ShapeDtypeStruct((_NC * NP, F), jnp.float32),
        mesh=_sc_mesh(),
        scratch_types=scratch,
        compiler_params=pltpu.CompilerParams(use_tc_tiling_on_sc=False),
    )
    def body(x_hbm, src_hbm, dst_hbm, adj_hbm, out_hbm, acc,
             src_v, dst_v, adj_v, rs, rd, zbuf, *tl):
        cid = lax.axis_index("c")
        sid = lax.axis_index("s")
        zero16 = jnp.zeros((_L,), jnp.float32)

        def zrow(r, carry):
            for j in range(nf):
                zbuf[r, pl.ds(j * _L, _L)] = zero16
            return carry
        lax.fori_loop(0, ZB, zrow, 0)
        for k in range(RPT // ZB):
            pltpu.sync_copy(zbuf, acc.at[pl.ds(sid * RPT + k * ZB, ZB)])
        plsc.subcore_barrier()

        ebase = cid * EC + sid * ET

        def chunk(off, sv, dv, av, rsv, rdv, n):
            pltpu.sync_copy(src_hbm.at[pl.ds(off, n)], sv)
            pltpu.sync_copy(dst_hbm.at[pl.ds(off, n)], dv)
            pltpu.sync_copy(adj_hbm.at[pl.ds(off, n)], av.at[pl.ds(0, n)])
            pltpu.sync_copy(x_hbm.at[sv], rsv)
            pltpu.sync_copy(x_hbm.at[dv], rdv)

            def blockfn(e, avec):
                for j in range(nf):
                    sl = pl.ds(j * _L, _L)
                    d = rsv[e, sl] - rdv[e, sl]
                    rsv[e, sl] = d * d * avec
            _scale_groups(av, n, blockfn)
            pltpu.sync_copy(rsv, acc.at[sv], add=True)

        def main_loop(i, carry):
            chunk(ebase + i * _CH, src_v, dst_v, adj_v, rs, rd, _CH)
            return carry
        lax.fori_loop(0, nfull, main_loop, 0)
        if tail:
            chunk(ebase + nfull * _CH, tl[0], tl[1], tl[2], tl[3], tl[4], tail)
        plsc.subcore_barrier()

        out_base = cid * NP + sid * RPT
        for k in range(RPT // ZB):
            pltpu.sync_copy(acc.at[pl.ds(sid * RPT + k * ZB, ZB)], zbuf)
            pltpu.sync_copy(zbuf, out_hbm.at[pl.ds(out_base + k * ZB, ZB)])

    return body(x, src, dst, adj)


def _propagate_sc(y, src, dst, adj):
    N, G = y.shape
    E = src.shape[0]
    EC = E // _NC
    ET = EC // _NS
    nfull = ET // _CH
    tail = ET % _CH
    NP = -(-N // (_NS * 128)) * (_NS * 128)
    RPT = NP // _NS
    ZB = _zero_chunk_rows(RPT)
    ng = G // _L

    scratch = [
        pltpu.VMEM_SHARED((NP, G), jnp.float32),
        pltpu.VMEM((_CH,), jnp.int32),
        pltpu.VMEM((_CH,), jnp.int32),
        pltpu.VMEM((_CH,), jnp.float32),
        pltpu.VMEM((_CH, G), jnp.float32),
        pltpu.VMEM((ZB, G), jnp.float32),
    ]
    if tail:
        tpad = ((tail + _L - 1) // _L) * _L
        scratch += [
            pltpu.VMEM((tail,), jnp.int32),
            pltpu.VMEM((tail,), jnp.int32),
            pltpu.VMEM((tpad,), jnp.float32),
            pltpu.VMEM((tail, G), jnp.float32),
        ]

    @functools.partial(
        pl.kernel,
        out_type=jax.ShapeDtypeStruct((_NC * NP, G), jnp.float32),
        mesh=_sc_mesh(),
        scratch_types=scratch,
        compiler_params=pltpu.CompilerParams(use_tc_tiling_on_sc=False),
    )
    def body(y_hbm, src_hbm, dst_hbm, adj_hbm, out_hbm, acc,
             src_v, dst_v, adj_v, rows, zbuf, *tl):
        cid = lax.axis_index("c")
        sid = lax.axis_index("s")
        zero16 = jnp.zeros((_L,), jnp.float32)

        def zrow(r, carry):
            for j in range(ng):
                zbuf[r, pl.ds(j * _L, _L)] = zero16
            return carry
        lax.fori_loop(0, ZB, zrow, 0)
        for k in range(RPT // ZB):
            pltpu.sync_copy(zbuf, acc.at[pl.ds(sid * RPT + k * ZB, ZB)])
        plsc.subcore_barrier()

        ebase = cid * EC + sid * ET

        def chunk(off, sv, dv, av, rv, n):
            pltpu.sync_copy(src_hbm.at[pl.ds(off, n)], sv)
            pltpu.sync_copy(dst_hbm.at[pl.ds(off, n)], dv)
            pltpu.sync_copy(adj_hbm.at[pl.ds(off, n)], av.at[pl.ds(0, n)])
            pltpu.sync_copy(y_hbm.at[dv], rv)

            def blockfn(e, avec):
                for j in range(ng):
                    sl = pl.ds(j * _L, _L)
                    rv[e, sl] = rv[e, sl] * avec
            _scale_groups(av, n, blockfn)
            pltpu.sync_copy(rv, acc.at[sv], add=True)

        def main_loop(i, carry):
            chunk(ebase + i * _CH, src_v, dst_v, adj_v, rows, _CH)
            return carry
        lax.fori_loop(0, nfull, main_loop, 0)
        if tail:
            chunk(ebase + nfull * _CH, tl[0], tl[1], tl[2], tl[3], tail)
        plsc.subcore_barrier()

        out_base = cid * NP + sid * RPT
        for k in range(RPT // ZB):
            pltpu.sync_copy(acc.at[pl.ds(sid * RPT + k * ZB, ZB)], zbuf)
            pltpu.sync_copy(zbuf, out_hbm.at[pl.ds(out_base + k * ZB, ZB)])

    return body(y, src, dst, adj)


def _fc_tc(x, m0, m1, degcol, sigma, W, b):
    N, F = x.shape
    G = W.shape[1]

    def body(x_ref, m0_ref, m1_ref, deg_ref, sig_ref, w_ref, b_ref, o_ref):
        sig = sig_ref[...]
        inv = 1.0 / (sig * sig)
        t = (m0_ref[...] + m1_ref[...]) * inv / deg_ref[...]
        xm = jnp.exp(-t) * x_ref[...]
        o_ref[...] = (
            jnp.dot(xm, w_ref[...], preferred_element_type=jnp.float32)
            + b_ref[...]
        )

    return pl.pallas_call(
        body, out_shape=jax.ShapeDtypeStruct((N, G), jnp.float32),
    )(x, m0, m1, degcol, sigma.reshape(1, F), W, b.reshape(1, G))


def _relu_combine_tc(p0, p1):
    def body(a_ref, b_ref, o_ref):
        o_ref[...] = jnp.maximum(a_ref[...] + b_ref[...], 0.0)

    return pl.pallas_call(
        body, out_shape=jax.ShapeDtypeStruct(p0.shape, jnp.float32),
    )(p0, p1)


def _log_softmax_tc(p0, p1, nclass):
    N, GP = p0.shape

    def body(a_ref, b_ref, o_ref):
        s = a_ref[...] + b_ref[...]
        col = lax.broadcasted_iota(jnp.int32, (N, GP), 1)
        valid = col < nclass
        s = jnp.where(valid, s, -1e30)
        m = jnp.max(s, axis=1, keepdims=True)
        e = jnp.where(valid, jnp.exp(s - m), 0.0)
        lse = jnp.log(jnp.sum(e, axis=1, keepdims=True))
        r = s - m - lse
        o_ref[...] = r[:, :nclass]

    return pl.pallas_call(
        body, out_shape=jax.ShapeDtypeStruct((N, nclass), jnp.float32),
    )(p0, p1)


def kernel(x, edge_index, adj_vals, deg, sigma1, W1, b1, sigma2, W2, b2):
    N, F = x.shape
    src = edge_index[0]
    dst = edge_index[1]
    degcol = deg.reshape(N, 1)
    nclass = W2.shape[1]
    gp = ((nclass + _L - 1) // _L) * _L
    W2p = jnp.pad(W2, ((0, 0), (0, gp - nclass)))
    b2p = jnp.pad(b2, (0, gp - nclass))
    NP = -(-N // (_NS * 128)) * (_NS * 128)

    m1p = _edge_mask_sc(x, src, dst, adj_vals)
    y1 = _fc_tc(x, m1p[:N], m1p[NP:NP + N], degcol, sigma1, W1, b1)
    o1p = _propagate_sc(y1, src, dst, adj_vals)
    h = _relu_combine_tc(o1p[:N], o1p[NP:NP + N])
    m2p = _edge_mask_sc(h, src, dst, adj_vals)
    y2 = _fc_tc(h, m2p[:N], m2p[NP:NP + N], degcol, sigma2, W2p, b2p)
    o2p = _propagate_sc(y2, src, dst, adj_vals)
    return _log_softmax_tc(o2p[:N], o2p[NP:NP + N], nclass)


# speedup vs baseline: 9.9319x; 1.2540x over previous
import functools

import jax
import jax.numpy as jnp
from jax import lax
from jax.experimental import pallas as pl
from jax.experimental.pallas import tpu as pltpu
from jax.experimental.pallas import tpu_sc as plsc

_NC = 2
_NS = 16
_L = 16
_CH = 128


def _zero_chunk_rows(rpt):
    for z in range(min(rpt, 64), 0, -1):
        if rpt % z == 0:
            return z
    return 1


def _lane_bcast(v16, lane):
    sel = jnp.full((_L,), lane, jnp.int32)
    return v16.at[sel].get(mode="promise_in_bounds")


def _scale_groups(av_ref, n, blockfn):
    gfull, rem = n // _L, n % _L

    def group(g, en):
        av16 = av_ref[pl.ds(g * _L, _L)]
        for e16 in range(en):
            blockfn(g * _L + e16, _lane_bcast(av16, e16))

    if gfull:
        def gbody(g, carry):
            group(g, _L)
            return carry
        lax.fori_loop(0, gfull, gbody, 0)
    if rem:
        group(gfull, rem)


def _sc_mesh():
    return plsc.VectorSubcoreMesh(core_axis_name="c", subcore_axis_name="s")


def _edge_mask_sc(x, src, dst, adj):
    N, F = x.shape
    E = src.shape[0]
    EC = E // _NC
    ET = EC // _NS
    nfull = ET // _CH
    tail = ET % _CH
    NP = -(-N // (_NS * 128)) * (_NS * 128)
    RPT = NP // _NS
    ZB = _zero_chunk_rows(RPT)
    nf = F // _L

    scratch = [
        pltpu.VMEM_SHARED((NP, F), jnp.float32),
        pltpu.VMEM((_CH,), jnp.int32),
        pltpu.VMEM((_CH,), jnp.int32),
        pltpu.VMEM((_CH,), jnp.float32),
        pltpu.VMEM((_CH, F), jnp.float32),
        pltpu.VMEM((_CH, F), jnp.float32),
        pltpu.VMEM((ZB, F), jnp.float32),
        pltpu.SemaphoreType.DMA,
        pltpu.SemaphoreType.DMA,
    ]
    if tail:
        tpad = ((tail + _L - 1) // _L) * _L
        scratch += [
            pltpu.VMEM((tail,), jnp.int32),
            pltpu.VMEM((tail,), jnp.int32),
            pltpu.VMEM((tpad,), jnp.float32),
            pltpu.VMEM((tail, F), jnp.float32),
            pltpu.VMEM((tail, F), jnp.float32),
        ]

    @functools.partial(
        pl.kernel,
        out_type=jax.ShapeDtypeStruct((_NC * NP, F), jnp.float32),
        mesh=_sc_mesh(),
        scratch_types=scratch,
        compiler_params=pltpu.CompilerParams(use_tc_tiling_on_sc=False),
    )
    def body(x_hbm, src_hbm, dst_hbm, adj_hbm, out_hbm, acc,
             src_v, dst_v, adj_v, rs, rd, zbuf, isem, gsem, *tl):
        cid = lax.axis_index("c")
        sid = lax.axis_index("s")
        zero16 = jnp.zeros((_L,), jnp.float32)

        def zrow(r, carry):
            for j in range(nf):
                zbuf[r, pl.ds(j * _L, _L)] = zero16
            return carry
        lax.fori_loop(0, ZB, zrow, 0)
        for k in range(RPT // ZB):
            pltpu.sync_copy(zbuf, acc.at[pl.ds(sid * RPT + k * ZB, ZB)])
        plsc.subcore_barrier()

        ebase = cid * EC + sid * ET

        def chunk(off, sv, dv, av, rsv, rdv, n):
            pltpu.async_copy(src_hbm.at[pl.ds(off, n)], sv, isem)
            pltpu.async_copy(dst_hbm.at[pl.ds(off, n)], dv, isem)
            pltpu.async_copy(adj_hbm.at[pl.ds(off, n)], av.at[pl.ds(0, n)], isem)
            pltpu.make_async_copy(src_hbm.at[pl.ds(off, n)], sv, isem).wait()
            pltpu.make_async_copy(dst_hbm.at[pl.ds(off, n)], dv, isem).wait()
            pltpu.make_async_copy(
                adj_hbm.at[pl.ds(off, n)], av.at[pl.ds(0, n)], isem).wait()
            pltpu.async_copy(x_hbm.at[sv], rsv, gsem)
            pltpu.async_copy(x_hbm.at[dv], rdv, gsem)
            pltpu.make_async_copy(x_hbm.at[sv], rsv, gsem).wait()
            pltpu.make_async_copy(x_hbm.at[dv], rdv, gsem).wait()

            def blockfn(e, avec):
                for j in range(nf):
                    sl = pl.ds(j * _L, _L)
                    d = rsv[e, sl] - rdv[e, sl]
                    rsv[e, sl] = d * d * avec
            _scale_groups(av, n, blockfn)
            pltpu.sync_copy(rsv, acc.at[sv], add=True)

        def main_loop(i, carry):
            chunk(ebase + i * _CH, src_v, dst_v, adj_v, rs, rd, _CH)
            return carry
        lax.fori_loop(0, nfull, main_loop, 0)
        if tail:
            chunk(ebase + nfull * _CH, tl[0], tl[1], tl[2], tl[3], tl[4], tail)
        plsc.subcore_barrier()

        out_base = cid * NP + sid * RPT
        for k in range(RPT // ZB):
            pltpu.sync_copy(acc.at[pl.ds(sid * RPT + k * ZB, ZB)], zbuf)
            pltpu.sync_copy(zbuf, out_hbm.at[pl.ds(out_base + k * ZB, ZB)])

    return body(x, src, dst, adj)


def _propagate_sc(y, src, dst, adj):
    N, G = y.shape
    E = src.shape[0]
    EC = E // _NC
    ET = EC // _NS
    nfull = ET // _CH
    tail = ET % _CH
    NP = -(-N // (_NS * 128)) * (_NS * 128)
    RPT = NP // _NS
    ZB = _zero_chunk_rows(RPT)
    ng = G // _L

    scratch = [
        pltpu.VMEM_SHARED((NP, G), jnp.float32),
        pltpu.VMEM((_CH,), jnp.int32),
        pltpu.VMEM((_CH,), jnp.int32),
        pltpu.VMEM((_CH,), jnp.float32),
        pltpu.VMEM((_CH, G), jnp.float32),
        pltpu.VMEM((ZB, G), jnp.float32),
        pltpu.SemaphoreType.DMA,
    ]
    if tail:
        tpad = ((tail + _L - 1) // _L) * _L
        scratch += [
            pltpu.VMEM((tail,), jnp.int32),
            pltpu.VMEM((tail,), jnp.int32),
            pltpu.VMEM((tpad,), jnp.float32),
            pltpu.VMEM((tail, G), jnp.float32),
        ]

    @functools.partial(
        pl.kernel,
        out_type=jax.ShapeDtypeStruct((_NC * NP, G), jnp.float32),
        mesh=_sc_mesh(),
        scratch_types=scratch,
        compiler_params=pltpu.CompilerParams(use_tc_tiling_on_sc=False),
    )
    def body(y_hbm, src_hbm, dst_hbm, adj_hbm, out_hbm, acc,
             src_v, dst_v, adj_v, rows, zbuf, isem, *tl):
        cid = lax.axis_index("c")
        sid = lax.axis_index("s")
        zero16 = jnp.zeros((_L,), jnp.float32)

        def zrow(r, carry):
            for j in range(ng):
                zbuf[r, pl.ds(j * _L, _L)] = zero16
            return carry
        lax.fori_loop(0, ZB, zrow, 0)
        for k in range(RPT // ZB):
            pltpu.sync_copy(zbuf, acc.at[pl.ds(sid * RPT + k * ZB, ZB)])
        plsc.subcore_barrier()

        ebase = cid * EC + sid * ET

        def chunk(off, sv, dv, av, rv, n):
            pltpu.async_copy(src_hbm.at[pl.ds(off, n)], sv, isem)
            pltpu.async_copy(dst_hbm.at[pl.ds(off, n)], dv, isem)
            pltpu.async_copy(adj_hbm.at[pl.ds(off, n)], av.at[pl.ds(0, n)], isem)
            pltpu.make_async_copy(src_hbm.at[pl.ds(off, n)], sv, isem).wait()
            pltpu.make_async_copy(dst_hbm.at[pl.ds(off, n)], dv, isem).wait()
            pltpu.make_async_copy(
                adj_hbm.at[pl.ds(off, n)], av.at[pl.ds(0, n)], isem).wait()
            pltpu.sync_copy(y_hbm.at[dv], rv)

            def blockfn(e, avec):
                for j in range(ng):
                    sl = pl.ds(j * _L, _L)
                    rv[e, sl] = rv[e, sl] * avec
            _scale_groups(av, n, blockfn)
            pltpu.sync_copy(rv, acc.at[sv], add=True)

        def main_loop(i, carry):
            chunk(ebase + i * _CH, src_v, dst_v, adj_v, rows, _CH)
            return carry
        lax.fori_loop(0, nfull, main_loop, 0)
        if tail:
            chunk(ebase + nfull * _CH, tl[0], tl[1], tl[2], tl[3], tail)
        plsc.subcore_barrier()

        out_base = cid * NP + sid * RPT
        for k in range(RPT // ZB):
            pltpu.sync_copy(acc.at[pl.ds(sid * RPT + k * ZB, ZB)], zbuf)
            pltpu.sync_copy(zbuf, out_hbm.at[pl.ds(out_base + k * ZB, ZB)])

    return body(y, src, dst, adj)


def _fc_tc(x, m0, m1, degcol, sigma, W, b):
    N, F = x.shape
    G = W.shape[1]

    def body(x_ref, m0_ref, m1_ref, deg_ref, sig_ref, w_ref, b_ref, o_ref):
        sig = sig_ref[...]
        inv = 1.0 / (sig * sig)
        t = (m0_ref[...] + m1_ref[...]) * inv / deg_ref[...]
        xm = jnp.exp(-t) * x_ref[...]
        o_ref[...] = (
            jnp.dot(xm, w_ref[...], preferred_element_type=jnp.float32)
            + b_ref[...]
        )

    return pl.pallas_call(
        body, out_shape=jax.ShapeDtypeStruct((N, G), jnp.float32),
    )(x, m0, m1, degcol, sigma.reshape(1, F), W, b.reshape(1, G))


def _relu_combine_tc(p0, p1):
    def body(a_ref, b_ref, o_ref):
        o_ref[...] = jnp.maximum(a_ref[...] + b_ref[...], 0.0)

    return pl.pallas_call(
        body, out_shape=jax.ShapeDtypeStruct(p0.shape, jnp.float32),
    )(p0, p1)


def _log_softmax_tc(p0, p1, nclass):
    N, GP = p0.shape

    def body(a_ref, b_ref, o_ref):
        s = a_ref[...] + b_ref[...]
        col = lax.broadcasted_iota(jnp.int32, (N, GP), 1)
        valid = col < nclass
        s = jnp.where(valid, s, -1e30)
        m = jnp.max(s, axis=1, keepdims=True)
        e = jnp.where(valid, jnp.exp(s - m), 0.0)
        lse = jnp.log(jnp.sum(e, axis=1, keepdims=True))
        r = s - m - lse
        o_ref[...] = r[:, :nclass]

    return pl.pallas_call(
        body, out_shape=jax.ShapeDtypeStruct((N, nclass), jnp.float32),
    )(p0, p1)


def kernel(x, edge_index, adj_vals, deg, sigma1, W1, b1, sigma2, W2, b2):
    N, F = x.shape
    src = edge_index[0]
    dst = edge_index[1]
    degcol = deg.reshape(N, 1)
    nclass = W2.shape[1]
    gp = ((nclass + _L - 1) // _L) * _L
    W2p = jnp.pad(W2, ((0, 0), (0, gp - nclass)))
    b2p = jnp.pad(b2, (0, gp - nclass))
    NP = -(-N // (_NS * 128)) * (_NS * 128)

    m1p = _edge_mask_sc(x, src, dst, adj_vals)
    y1 = _fc_tc(x, m1p[:N], m1p[NP:NP + N], degcol, sigma1, W1, b1)
    o1p = _propagate_sc(y1, src, dst, adj_vals)
    h = _relu_combine_tc(o1p[:N], o1p[NP:NP + N])
    m2p = _edge_mask_sc(h, src, dst, adj_vals)
    y2 = _fc_tc(h, m2p[:N], m2p[NP:NP + N], degcol, sigma2, W2p, b2p)
    o2p = _propagate_sc(y2, src, dst, adj_vals)
    return _log_softmax_tc(o2p[:N], o2p[NP:NP + N], nclass)
